# baseline (speedup 1.0000x reference)
import functools

import jax
import jax.numpy as jnp
from jax import lax
from jax.experimental import pallas as pl
from jax.experimental.pallas import tpu as pltpu
from jax.experimental.pallas import tpu_sc as plsc

B, S = 256, 512
EMBED = 768
D4 = EMBED // 4
D2 = EMBED // 2
NF, NT = 8, 64
NFI, NTI = 2 * NF - 1, 2 * NT - 1
NPOS = B * S
NC, NS = 2, 16
NW = NC * NS
ROWS_PER_W = B // NW
CHUNK = 32
NCHUNK = S // CHUNK
AROW = D2 + D4
CB = CHUNK * EMBED
L = 16


def _make_sc_kernel():
    mesh = plsc.VectorSubcoreMesh(core_axis_name="c", subcore_axis_name="s")

    @functools.partial(
        pl.kernel,
        mesh=mesh,
        out_type=jax.ShapeDtypeStruct((NPOS * EMBED,), jnp.float32),
        compiler_params=pltpu.CompilerParams(needs_layout_passes=False),
        scratch_types=[
            pltpu.VMEM((S,), jnp.int32),
            pltpu.VMEM((S,), jnp.int32),
            pltpu.VMEM((S,), jnp.int32),
            pltpu.VMEM((S,), jnp.int32),
            pltpu.VMEM((S,), jnp.float32),
            pltpu.VMEM((16 * AROW,), jnp.float32),
            pltpu.VMEM((128 * D4,), jnp.float32),
            pltpu.VMEM((2, CB), jnp.float32),
            pltpu.VMEM((3 * D2,), jnp.float32),
            pltpu.VMEM((L,), jnp.float32),
            pltpu.SemaphoreType.DMA,
            pltpu.SemaphoreType.DMA,
        ],
    )
    def k(fp_hbm, tp_hbm, tf_hbm, tt_hbm, w_hbm, out_hbm,
          fpb, tpb, aidx, tidx, rtv, ab, tb, stage, wbuf, redb, ws0, ws1):
        wid = lax.axis_index("s") * NC + lax.axis_index("c")
        pltpu.sync_copy(w_hbm, wbuf)
        pltpu.sync_copy(tt_hbm, tb)
        for kk in range(NFI):
            pltpu.sync_copy(tf_hbm.at[pl.ds(kk * D4, D4)],
                            ab.at[pl.ds(kk * AROW + D2, D4)])
        lanes = lax.broadcasted_iota(jnp.int32, (L,), 0)

        def lane_total(v):
            for step in (1, 2, 4, 8):
                redb[...] = v
                v = v + plsc.load_gather(redb, [jnp.bitwise_xor(lanes, step)])
            return v

        w0 = [wbuf[pl.ds(L * j, L)] for j in range(D2 // L)]
        w1 = [wbuf[pl.ds(D2 + L * j, L)] for j in range(D2 // L)]
        bv = [wbuf[pl.ds(2 * D2 + L * j, L)] for j in range(D2 // L)]

        def row_body(r, carry):
            base = (wid * ROWS_PER_W + r) * S
            pltpu.sync_copy(fp_hbm.at[pl.ds(base, S)], fpb)
            pltpu.sync_copy(tp_hbm.at[pl.ds(base, S)], tpb)

            def sum_body(g, accs):
                fa, ta = accs
                fa = fa + fpb[pl.ds(L * g, L)].astype(jnp.float32)
                ta = ta + tpb[pl.ds(L * g, L)].astype(jnp.float32)
                return fa, ta

            facc, tacc = lax.fori_loop(
                0, S // L, sum_body,
                (jnp.zeros((L,), jnp.float32), jnp.zeros((L,), jnp.float32)))
            fc = lane_total(facc) * (1.0 / S)
            tc = lane_total(tacc) * (1.0 / S)
            af = (NF - 1) - fc
            at = (NT - 1) - tc
            kf = af.astype(jnp.int32)
            kt = at.astype(jnp.int32)
            df = af - kf.astype(jnp.float32)

            def idx_body(g, c2):
                fv = fpb[pl.ds(L * g, L)]
                tv = tpb[pl.ds(L * g, L)]
                aidx[pl.ds(L * g, L)] = (fv + kf) * AROW
                tidx[pl.ds(L * g, L)] = (tv + kt) * D4
                rtv[pl.ds(L * g, L)] = tv.astype(jnp.float32) - tc
                return c2

            lax.fori_loop(0, S // L, idx_body, 0)

            def abuild_body(kk, c2):
                coef = (kk - (NF - 1)).astype(jnp.float32) + df
                kbase = kk * AROW
                for j in range(D2 // L):
                    ab[pl.ds(kbase + L * j, L)] = coef * w0[j] + bv[j]
                return c2

            lax.fori_loop(0, 16, abuild_body, 0)

            def assemble(c, buf):
                def pos_body(p, c2):
                    pvec = jnp.broadcast_to(c * CHUNK + p, (L,)).astype(jnp.int32)
                    av = plsc.load_gather(aidx, [pvec])
                    tv2 = plsc.load_gather(tidx, [pvec])
                    rt = plsc.load_gather(rtv, [pvec])
                    sbase = p * EMBED
                    for j in range(AROW // L):
                        x = plsc.load_gather(ab, [av + (lanes + L * j)])
                        if j < D2 // L:
                            stage[buf, pl.ds(sbase + L * j, L)] = x + rt * w1[j]
                        else:
                            stage[buf, pl.ds(sbase + L * j, L)] = x
                    for j in range(D4 // L):
                        x = plsc.load_gather(tb, [tv2 + (lanes + L * j)])
                        stage[buf, pl.ds(sbase + AROW + L * j, L)] = x
                    return c2

                lax.fori_loop(0, CHUNK, pos_body, 0)

            def write(c, buf, sem):
                dst = out_hbm.at[pl.ds((base + c * CHUNK) * EMBED, CB)]
                pltpu.async_copy(stage.at[buf], dst, sem)

            def write_wait(c, buf, sem):
                dst = out_hbm.at[pl.ds((base + c * CHUNK) * EMBED, CB)]
                pltpu.make_async_copy(stage.at[buf], dst, sem).wait()

            def pair_body(c2, carry2):
                c0 = 2 * c2

                @pl.when(c2 > 0)
                def _():
                    write_wait(c0 - 2, 0, ws0)

                assemble(c0, 0)
                write(c0, 0, ws0)

                @pl.when(c2 > 0)
                def _():
                    write_wait(c0 - 1, 1, ws1)

                assemble(c0 + 1, 1)
                write(c0 + 1, 1, ws1)
                return carry2

            lax.fori_loop(0, NCHUNK // 2, pair_body, 0)
            write_wait(NCHUNK - 2, 0, ws0)
            write_wait(NCHUNK - 1, 1, ws1)
            return carry

        lax.fori_loop(0, ROWS_PER_W, row_body, 0)

    return k


_sc_call = _make_sc_kernel()


def kernel(freq_positions, time_positions, freq_relative_emb, time_relative_emb, W_dist, b_dist):
    fp = freq_positions.reshape(-1).astype(jnp.int32)
    tp = time_positions.reshape(-1).astype(jnp.int32)
    tf_flat = freq_relative_emb.reshape(-1)
    tt_flat = jnp.pad(time_relative_emb, ((0, 1), (0, 0))).reshape(-1)
    wflat = jnp.concatenate([W_dist[0], W_dist[1], b_dist])
    out = _sc_call(fp, tp, tf_flat, tt_flat, wflat)
    return out.reshape(B, S, EMBED)


# speedup vs baseline: 1.7554x; 1.7554x over previous
import functools

import jax
import jax.numpy as jnp
from jax import lax
from jax.experimental import pallas as pl
from jax.experimental.pallas import tpu as pltpu
from jax.experimental.pallas import tpu_sc as plsc

B, S = 256, 512
EMBED = 768
D4 = EMBED // 4
D2 = EMBED // 2
NF, NT = 8, 64
NFI, NTI = 2 * NF - 1, 2 * NT - 1
NPOS = B * S
NC, NS = 2, 16
NW = NC * NS
ROWS_PER_W = B // NW
CHUNK = 32
NCHUNK = S // CHUNK
AROW = D2 + D4
CB = CHUNK * EMBED
L = 16


def _make_sc_kernel():
    mesh = plsc.VectorSubcoreMesh(core_axis_name="c", subcore_axis_name="s")

    @functools.partial(
        pl.kernel,
        mesh=mesh,
        out_type=jax.ShapeDtypeStruct((NPOS * EMBED,), jnp.float32),
        compiler_params=pltpu.CompilerParams(needs_layout_passes=False),
        scratch_types=[
            pltpu.VMEM((S,), jnp.int32),
            pltpu.VMEM((S,), jnp.int32),
            pltpu.VMEM((S,), jnp.int32),
            pltpu.VMEM((S,), jnp.int32),
            pltpu.VMEM((S,), jnp.float32),
            pltpu.VMEM((16 * AROW,), jnp.float32),
            pltpu.VMEM((128 * D4,), jnp.float32),
            pltpu.VMEM((2, CB), jnp.float32),
            pltpu.VMEM((3 * D2,), jnp.float32),
            pltpu.VMEM((L,), jnp.float32),
            pltpu.SemaphoreType.DMA,
            pltpu.SemaphoreType.DMA,
        ],
    )
    def k(fp_hbm, tp_hbm, tf_hbm, tt_hbm, w_hbm, out_hbm,
          fpb, tpb, aidx, tidx, rtv, ab, tb, stage, wbuf, redb, ws0, ws1):
        wid = lax.axis_index("s") * NC + lax.axis_index("c")
        pltpu.sync_copy(w_hbm, wbuf)
        pltpu.sync_copy(tt_hbm, tb)
        for kk in range(NFI):
            pltpu.sync_copy(tf_hbm.at[pl.ds(kk * D4, D4)],
                            ab.at[pl.ds(kk * AROW + D2, D4)])
        lanes = lax.broadcasted_iota(jnp.int32, (L,), 0)

        def lane_total(v):
            for step in (1, 2, 4, 8):
                redb[...] = v
                v = v + plsc.load_gather(redb, [jnp.bitwise_xor(lanes, step)])
            return v

        w0 = [wbuf[pl.ds(L * j, L)] for j in range(D2 // L)]
        w1 = [wbuf[pl.ds(D2 + L * j, L)] for j in range(D2 // L)]
        bv = [wbuf[pl.ds(2 * D2 + L * j, L)] for j in range(D2 // L)]

        def row_body(r, carry):
            base = (wid * ROWS_PER_W + r) * S
            pltpu.sync_copy(fp_hbm.at[pl.ds(base, S)], fpb)
            pltpu.sync_copy(tp_hbm.at[pl.ds(base, S)], tpb)

            def sum_body(g, accs):
                fa, ta = accs
                fa = fa + fpb[pl.ds(L * g, L)].astype(jnp.float32)
                ta = ta + tpb[pl.ds(L * g, L)].astype(jnp.float32)
                return fa, ta

            facc, tacc = lax.fori_loop(
                0, S // L, sum_body,
                (jnp.zeros((L,), jnp.float32), jnp.zeros((L,), jnp.float32)))
            fc = lane_total(facc) * (1.0 / S)
            tc = lane_total(tacc) * (1.0 / S)
            af = (NF - 1) - fc
            at = (NT - 1) - tc
            kf = af.astype(jnp.int32)
            kt = at.astype(jnp.int32)
            df = af - kf.astype(jnp.float32)

            @plsc.parallel_loop(0, S // L)
            def _idx_body(g):
                fv = fpb[pl.ds(L * g, L)]
                tv = tpb[pl.ds(L * g, L)]
                aidx[pl.ds(L * g, L)] = (fv + kf) * AROW
                tidx[pl.ds(L * g, L)] = (tv + kt) * D4
                rtv[pl.ds(L * g, L)] = tv.astype(jnp.float32) - tc

            @plsc.parallel_loop(0, 16)
            def _abuild_body(kk):
                coef = (kk - (NF - 1)).astype(jnp.float32) + df
                kbase = kk * AROW
                for j in range(D2 // L):
                    ab[pl.ds(kbase + L * j, L)] = coef * w0[j] + bv[j]

            def assemble(c, buf):
                @plsc.parallel_loop(0, CHUNK, unroll=2)
                def _pos_body(p):
                    pvec = jnp.broadcast_to(c * CHUNK + p, (L,)).astype(jnp.int32)
                    av = plsc.load_gather(aidx, [pvec])
                    tv2 = plsc.load_gather(tidx, [pvec])
                    rt = plsc.load_gather(rtv, [pvec])
                    sbase = p * EMBED
                    for j in range(AROW // L):
                        x = plsc.load_gather(ab, [av + (lanes + L * j)])
                        if j < D2 // L:
                            stage[buf, pl.ds(sbase + L * j, L)] = x + rt * w1[j]
                        else:
                            stage[buf, pl.ds(sbase + L * j, L)] = x
                    for j in range(D4 // L):
                        x = plsc.load_gather(tb, [tv2 + (lanes + L * j)])
                        stage[buf, pl.ds(sbase + AROW + L * j, L)] = x

            def write(c, buf, sem):
                dst = out_hbm.at[pl.ds((base + c * CHUNK) * EMBED, CB)]
                pltpu.async_copy(stage.at[buf], dst, sem)

            def write_wait(c, buf, sem):
                dst = out_hbm.at[pl.ds((base + c * CHUNK) * EMBED, CB)]
                pltpu.make_async_copy(stage.at[buf], dst, sem).wait()

            def pair_body(c2, carry2):
                c0 = 2 * c2

                @pl.when(c2 > 0)
                def _():
                    write_wait(c0 - 2, 0, ws0)

                assemble(c0, 0)
                write(c0, 0, ws0)

                @pl.when(c2 > 0)
                def _():
                    write_wait(c0 - 1, 1, ws1)

                assemble(c0 + 1, 1)
                write(c0 + 1, 1, ws1)
                return carry2

            lax.fori_loop(0, NCHUNK // 2, pair_body, 0)
            write_wait(NCHUNK - 2, 0, ws0)
            write_wait(NCHUNK - 1, 1, ws1)
            return carry

        lax.fori_loop(0, ROWS_PER_W, row_body, 0)

    return k


_sc_call = _make_sc_kernel()


def kernel(freq_positions, time_positions, freq_relative_emb, time_relative_emb, W_dist, b_dist):
    fp = freq_positions.reshape(-1).astype(jnp.int32)
    tp = time_positions.reshape(-1).astype(jnp.int32)
    tf_flat = freq_relative_emb.reshape(-1)
    tt_flat = jnp.pad(time_relative_emb, ((0, 1), (0, 0))).reshape(-1)
    wflat = jnp.concatenate([W_dist[0], W_dist[1], b_dist])
    out = _sc_call(fp, tp, tf_flat, tt_flat, wflat)
    return out.reshape(B, S, EMBED)


# speedup vs baseline: 4.1424x; 2.3598x over previous
import functools

import jax
import jax.numpy as jnp
from jax import lax
from jax.experimental import pallas as pl
from jax.experimental.pallas import tpu as pltpu
from jax.experimental.pallas import tpu_sc as plsc

B, S = 256, 512
EMBED = 768
D4 = EMBED // 4
D2 = EMBED // 2
NF, NT = 8, 64
NFI, NTI = 2 * NF - 1, 2 * NT - 1
NPOS = B * S
NC, NS = 2, 16
NW = NC * NS
ROWS_PER_W = B // NW
CHUNK = 32
NCHUNK = S // CHUNK
NBUF = 4
L = 16


def _make_sc_kernel():
    mesh = plsc.VectorSubcoreMesh(core_axis_name="c", subcore_axis_name="s")

    @functools.partial(
        pl.kernel,
        mesh=mesh,
        out_type=jax.ShapeDtypeStruct((NPOS, EMBED), jnp.float32),
        compiler_params=pltpu.CompilerParams(needs_layout_passes=False),
        scratch_types=[
            pltpu.VMEM((S,), jnp.int32),
            pltpu.VMEM((S,), jnp.int32),
            pltpu.VMEM((NCHUNK, CHUNK), jnp.int32),
            pltpu.VMEM((NBUF, CHUNK, EMBED), jnp.float32),
            pltpu.VMEM((2 * D2,), jnp.float32),
            pltpu.VMEM((D2,), jnp.float32),
            pltpu.VMEM((L,), jnp.float32),
            [pltpu.SemaphoreType.DMA] * NBUF,
            [pltpu.SemaphoreType.DMA] * NBUF,
        ],
    )
    def k(fp_hbm, tp_hbm, full_hbm, w_hbm, out_hbm,
          fpb, tpb, jidx, stage, wbuf, rbuf, redb, gsems, wsems):
        wid = lax.axis_index("s") * NC + lax.axis_index("c")
        pltpu.sync_copy(w_hbm, wbuf)
        lanes = lax.broadcasted_iota(jnp.int32, (L,), 0)

        def lane_total(v):
            for step in (1, 2, 4, 8):
                redb[...] = v
                v = v + plsc.load_gather(redb, [jnp.bitwise_xor(lanes, step)])
            return v

        def row_body(r, carry):
            base = (wid * ROWS_PER_W + r) * S
            pltpu.sync_copy(fp_hbm.at[pl.ds(base, S)], fpb)
            pltpu.sync_copy(tp_hbm.at[pl.ds(base, S)], tpb)
            facc = jnp.zeros((L,), jnp.float32)
            tacc = jnp.zeros((L,), jnp.float32)
            for g in range(S // L):
                facc = facc + fpb[pl.ds(L * g, L)].astype(jnp.float32)
                tacc = tacc + tpb[pl.ds(L * g, L)].astype(jnp.float32)
            fc = lane_total(facc) * (1.0 / S)
            tc = lane_total(tacc) * (1.0 / S)
            af = (NF - 1) - fc
            at = (NT - 1) - tc
            kf = af.astype(jnp.int32)
            kt = at.astype(jnp.int32)
            df = af - kf.astype(jnp.float32)
            dt = at - kt.astype(jnp.float32)
            off = kf * NTI + kt
            for g in range(S // L):
                fv = fpb[pl.ds(L * g, L)]
                tv = tpb[pl.ds(L * g, L)]
                jidx[g // (CHUNK // L), pl.ds((g % (CHUNK // L)) * L, L)] = (
                    fv * NTI + tv + off)
            for j in range(D2 // L):
                rbuf[pl.ds(L * j, L)] = (df * wbuf[pl.ds(L * j, L)]
                                         + dt * wbuf[pl.ds(D2 + L * j, L)])

            rv = [rbuf[pl.ds(L * j, L)] for j in range(D2 // L)]

            def add_bias(buf):
                for p in range(CHUNK):
                    for j in range(D2 // L):
                        plsc.addupdate(stage.at[buf, p, pl.ds(L * j, L)], rv[j])

            def gather(c, buf):
                pltpu.async_copy(full_hbm.at[jidx.at[c]], stage.at[buf], gsems[buf])

            def gather_wait(c, buf):
                pltpu.make_async_copy(full_hbm.at[jidx.at[c]], stage.at[buf],
                                      gsems[buf]).wait()

            def write(c, buf):
                dst = out_hbm.at[pl.ds(base + c * CHUNK, CHUNK)]
                pltpu.async_copy(stage.at[buf], dst, wsems[buf])

            def write_wait(c, buf):
                dst = out_hbm.at[pl.ds(base + c * CHUNK, CHUNK)]
                pltpu.make_async_copy(stage.at[buf], dst, wsems[buf]).wait()

            for b in range(NBUF):
                gather(b, b)

            def quad_body(q, carry2):
                c0 = NBUF * q
                for b in range(3):
                    gather_wait(c0 + b, b)
                    add_bias(b)
                    write(c0 + b, b)
                write_wait(c0, 0)

                @pl.when(c0 + NBUF < NCHUNK)
                def _():
                    gather(c0 + NBUF, 0)

                gather_wait(c0 + 3, 3)
                add_bias(3)
                write(c0 + 3, 3)
                for b in range(1, NBUF):
                    write_wait(c0 + b, b)

                    @pl.when(c0 + NBUF + b < NCHUNK)
                    def _():
                        gather(c0 + NBUF + b, b)

                return carry2

            lax.fori_loop(0, NCHUNK // NBUF, quad_body, 0)
            return carry

        lax.fori_loop(0, ROWS_PER_W, row_body, 0)

    return k


_sc_call = _make_sc_kernel()


def kernel(freq_positions, time_positions, freq_relative_emb, time_relative_emb, W_dist, b_dist):
    fp = freq_positions.reshape(-1).astype(jnp.int32)
    tp = time_positions.reshape(-1).astype(jnp.int32)
    vf = jnp.arange(NFI, dtype=jnp.float32) - (NF - 1)
    vt = jnp.arange(NTI, dtype=jnp.float32) - (NT - 1)
    dist = (vf[:, None, None] * W_dist[0][None, None, :]
            + vt[None, :, None] * W_dist[1][None, None, :]
            + b_dist[None, None, :])
    fpart = jnp.broadcast_to(freq_relative_emb[:, None, :], (NFI, NTI, D4))
    tpart = jnp.broadcast_to(time_relative_emb[None, :, :], (NFI, NTI, D4))
    full = jnp.concatenate([dist, fpart, tpart], axis=-1).reshape(NFI * NTI, EMBED)
    wflat = jnp.concatenate([W_dist[0], W_dist[1]])
    out = _sc_call(fp, tp, full, wflat)
    return out.reshape(B, S, EMBED)


# speedup vs baseline: 4.1443x; 1.0005x over previous
import functools

import jax
import jax.numpy as jnp
from jax import lax
from jax.experimental import pallas as pl
from jax.experimental.pallas import tpu as pltpu
from jax.experimental.pallas import tpu_sc as plsc

B, S = 256, 512
EMBED = 768
D4 = EMBED // 4
D2 = EMBED // 2
NF, NT = 8, 64
NFI, NTI = 2 * NF - 1, 2 * NT - 1
NPOS = B * S
NC, NS = 2, 16
NW = NC * NS
ROWS_PER_W = B // NW
CHUNK = 16
NCHUNK = S // CHUNK
NBUF = 8
L = 16


def _make_sc_kernel():
    mesh = plsc.VectorSubcoreMesh(core_axis_name="c", subcore_axis_name="s")

    @functools.partial(
        pl.kernel,
        mesh=mesh,
        out_type=jax.ShapeDtypeStruct((NPOS, EMBED), jnp.float32),
        compiler_params=pltpu.CompilerParams(needs_layout_passes=False),
        scratch_types=[
            pltpu.VMEM((S,), jnp.int32),
            pltpu.VMEM((S,), jnp.int32),
            pltpu.VMEM((NCHUNK, CHUNK), jnp.int32),
            pltpu.VMEM((NBUF, CHUNK, EMBED), jnp.float32),
            pltpu.VMEM((2 * D2,), jnp.float32),
            pltpu.VMEM((D2,), jnp.float32),
            pltpu.VMEM((L,), jnp.float32),
            [pltpu.SemaphoreType.DMA] * NBUF,
            [pltpu.SemaphoreType.DMA] * NBUF,
        ],
    )
    def k(fp_hbm, tp_hbm, full_hbm, w_hbm, out_hbm,
          fpb, tpb, jidx, stage, wbuf, rbuf, redb, gsems, wsems):
        wid = lax.axis_index("s") * NC + lax.axis_index("c")
        pltpu.sync_copy(w_hbm, wbuf)
        lanes = lax.broadcasted_iota(jnp.int32, (L,), 0)

        def lane_total(v):
            for step in (1, 2, 4, 8):
                redb[...] = v
                v = v + plsc.load_gather(redb, [jnp.bitwise_xor(lanes, step)])
            return v

        def row_body(r, carry):
            base = (wid * ROWS_PER_W + r) * S
            pltpu.sync_copy(fp_hbm.at[pl.ds(base, S)], fpb)
            pltpu.sync_copy(tp_hbm.at[pl.ds(base, S)], tpb)
            facc = jnp.zeros((L,), jnp.float32)
            tacc = jnp.zeros((L,), jnp.float32)
            for g in range(S // L):
                facc = facc + fpb[pl.ds(L * g, L)].astype(jnp.float32)
                tacc = tacc + tpb[pl.ds(L * g, L)].astype(jnp.float32)
            fc = lane_total(facc) * (1.0 / S)
            tc = lane_total(tacc) * (1.0 / S)
            af = (NF - 1) - fc
            at = (NT - 1) - tc
            kf = af.astype(jnp.int32)
            kt = at.astype(jnp.int32)
            df = af - kf.astype(jnp.float32)
            dt = at - kt.astype(jnp.float32)
            off = kf * NTI + kt
            for g in range(S // L):
                fv = fpb[pl.ds(L * g, L)]
                tv = tpb[pl.ds(L * g, L)]
                jidx[g // (CHUNK // L), pl.ds((g % (CHUNK // L)) * L, L)] = (
                    fv * NTI + tv + off)
            for j in range(D2 // L):
                rbuf[pl.ds(L * j, L)] = (df * wbuf[pl.ds(L * j, L)]
                                         + dt * wbuf[pl.ds(D2 + L * j, L)])

            rv = [rbuf[pl.ds(L * j, L)] for j in range(D2 // L)]

            def add_bias(buf):
                for p in range(CHUNK):
                    for j in range(D2 // L):
                        plsc.addupdate(stage.at[buf, p, pl.ds(L * j, L)], rv[j])

            def gather(c, buf):
                pltpu.async_copy(full_hbm.at[jidx.at[c]], stage.at[buf], gsems[buf])

            def gather_wait(c, buf):
                pltpu.make_async_copy(full_hbm.at[jidx.at[c]], stage.at[buf],
                                      gsems[buf]).wait()

            def write(c, buf):
                dst = out_hbm.at[pl.ds(base + c * CHUNK, CHUNK)]
                pltpu.async_copy(stage.at[buf], dst, wsems[buf])

            def write_wait(c, buf):
                dst = out_hbm.at[pl.ds(base + c * CHUNK, CHUNK)]
                pltpu.make_async_copy(stage.at[buf], dst, wsems[buf]).wait()

            for b in range(NBUF):
                gather(b, b)

            def quad_body(q, carry2):
                c0 = NBUF * q
                for b in range(NBUF - 1):
                    gather_wait(c0 + b, b)
                    add_bias(b)
                    write(c0 + b, b)
                write_wait(c0, 0)

                @pl.when(c0 + NBUF < NCHUNK)
                def _():
                    gather(c0 + NBUF, 0)

                gather_wait(c0 + NBUF - 1, NBUF - 1)
                add_bias(NBUF - 1)
                write(c0 + NBUF - 1, NBUF - 1)
                for b in range(1, NBUF):
                    write_wait(c0 + b, b)

                    @pl.when(c0 + NBUF + b < NCHUNK)
                    def _():
                        gather(c0 + NBUF + b, b)

                return carry2

            lax.fori_loop(0, NCHUNK // NBUF, quad_body, 0)
            return carry

        lax.fori_loop(0, ROWS_PER_W, row_body, 0)

    return k


_sc_call = _make_sc_kernel()


def kernel(freq_positions, time_positions, freq_relative_emb, time_relative_emb, W_dist, b_dist):
    fp = freq_positions.reshape(-1).astype(jnp.int32)
    tp = time_positions.reshape(-1).astype(jnp.int32)
    vf = jnp.arange(NFI, dtype=jnp.float32) - (NF - 1)
    vt = jnp.arange(NTI, dtype=jnp.float32) - (NT - 1)
    dist = (vf[:, None, None] * W_dist[0][None, None, :]
            + vt[None, :, None] * W_dist[1][None, None, :]
            + b_dist[None, None, :])
    fpart = jnp.broadcast_to(freq_relative_emb[:, None, :], (NFI, NTI, D4))
    tpart = jnp.broadcast_to(time_relative_emb[None, :, :], (NFI, NTI, D4))
    full = jnp.concatenate([dist, fpart, tpart], axis=-1).reshape(NFI * NTI, EMBED)
    wflat = jnp.concatenate([W_dist[0], W_dist[1]])
    out = _sc_call(fp, tp, full, wflat)
    return out.reshape(B, S, EMBED)
